# baseline (speedup 1.0000x reference)
import functools

import jax
import jax.numpy as jnp
from jax import lax
from jax.experimental import pallas as pl
from jax.experimental.pallas import tpu as pltpu
from jax.experimental.pallas import tpu_sc as plsc

_K = 8
_SQRT_HALF = 0.7071067811865476
_NC = 2
_NS = 16
_NW = _NC * _NS
_CH = 128


def _gelu(x):
    return 0.5 * x * (1.0 + lax.erf(x * jnp.float32(_SQRT_HALF)))


def _pack_bf16_pair(x):
    r = lax.bitcast_convert_type(x, jnp.uint32)
    r16 = (r + jnp.uint32(0x7FFF) + ((r >> 16) & jnp.uint32(1))) >> 16
    lo = r16[:, :128]
    hi = r16[:, 128:]
    return lax.bitcast_convert_type(lo | (hi << 16), jnp.int32)


def _unpack_bf16_pair(p):
    u = lax.bitcast_convert_type(p, jnp.uint32)
    f_lo = lax.bitcast_convert_type(u << 16, jnp.float32)
    f_hi = lax.bitcast_convert_type(u & jnp.uint32(0xFFFF0000), jnp.float32)
    return f_lo, f_hi


def _ctable_body(h_grid, Wp1_ref, bp1_ref, Wp2_ref, bp2_ref, Wc_ref, out_ref):
    T = out_ref.shape[0]
    S = 2 * h_grid - 1
    step = jnp.float32(1.0 / (h_grid - 1))
    Wc_comb = jnp.dot(Wp2_ref[...], Wc_ref[...], preferred_element_type=jnp.float32)
    bc_comb = jnp.dot(bp2_ref[...], Wc_ref[...], preferred_element_type=jnp.float32)
    d = lax.broadcasted_iota(jnp.int32, (T, 1), 0)
    dy = (d // S - (h_grid - 1)).astype(jnp.float32) * step
    dx = (d % S - (h_grid - 1)).astype(jnp.float32) * step
    g1 = _gelu(dy * Wp1_ref[0:1, :] + dx * Wp1_ref[1:2, :] + bp1_ref[...])
    ct = jnp.dot(g1, Wc_comb, preferred_element_type=jnp.float32) + bc_comb
    out_ref[...] = _pack_bf16_pair(ct)


def _main_body(h_grid, fm_ref, Wn_ref, bn_ref, g_ref, bln_ref, Wa_ref, Wb_ref,
               be1_ref, nodes_ref, adj_ref, gidx_ref, cidx_ref, a_ref, b_ref):
    N = fm_ref.shape[2]
    fm = fm_ref[0]
    h = lax.dot_general(fm, Wn_ref[...], (((0,), (0,)), ((), ())),
                        preferred_element_type=jnp.float32) + bn_ref[...]
    mu = jnp.mean(h, axis=-1, keepdims=True)
    var = jnp.mean((h - mu) ** 2, axis=-1, keepdims=True)
    h = (h - mu) / jnp.sqrt(var + 1e-5) * g_ref[...] + bln_ref[...]
    nodes = _gelu(h)
    nodes_ref[0] = nodes
    ss = jnp.sum(nodes * nodes, axis=-1, keepdims=True)
    norm = jnp.maximum(jnp.sqrt(ss), 1e-12)
    nrm = nodes / norm
    sim = lax.dot_general(nrm, nrm, (((1,), (1,)), ((), ())),
                          preferred_element_type=jnp.float32)
    rid = lax.broadcasted_iota(jnp.int32, (N, N), 0)
    cid = lax.broadcasted_iota(jnp.int32, (N, N), 1)
    sim = jnp.where(rid == cid, sim - 1e9, sim)
    cols = []
    s = sim
    for _ in range(_K):
        vmax = jnp.max(s, axis=-1, keepdims=True)
        idx = jnp.min(jnp.where(s == vmax, cid, N), axis=-1, keepdims=True)
        cols.append(idx)
        s = jnp.where(cid == idx, -jnp.inf, s)
    adj = jnp.concatenate(cols, axis=-1)
    adj_ref[0] = adj
    gidx_ref[0] = adj + pl.program_id(0) * N
    S = 2 * h_grid - 1
    rown = lax.broadcasted_iota(jnp.int32, (N, _K), 0)
    dyi = adj // h_grid - rown // h_grid + (h_grid - 1)
    dxi = adj % h_grid - rown % h_grid + (h_grid - 1)
    cidx_ref[0] = dyi * S + dxi
    a_ref[0] = jnp.dot(nodes, Wa_ref[...], preferred_element_type=jnp.float32) + be1_ref[...]
    b_ref[0] = _pack_bf16_pair(
        jnp.dot(nodes, Wb_ref[...], preferred_element_type=jnp.float32))


def _gather_body(n_chunks, btab_hbm, ctab_hbm, bidx_hbm, cidx_hbm,
                 nb_hbm, cc_hbm, bi0, bi1, ci0, ci1, br0, br1, cr0, cr1,
                 gsem0, gsem1, osem0, osem1):
    wid = lax.axis_index("s") * _NC + lax.axis_index("c")
    span = n_chunks * _CH
    bi = (bi0, bi1)
    ci = (ci0, ci1)
    br = (br0, br1)
    cr = (cr0, cr1)
    gsem = (gsem0, gsem1)
    osem = (osem0, osem1)

    def load_and_gather(j, p):
        base = wid * span + j * _CH
        pltpu.sync_copy(bidx_hbm.at[pl.ds(base, _CH)], bi[p])
        pltpu.sync_copy(cidx_hbm.at[pl.ds(base, _CH)], ci[p])
        gb = pltpu.async_copy(btab_hbm.at[bi[p]], br[p], gsem[p])
        gc = pltpu.async_copy(ctab_hbm.at[ci[p]], cr[p], gsem[p])
        return (gb, gc)

    def store_out(j, p):
        base = wid * span + j * _CH
        ob = pltpu.async_copy(br[p], nb_hbm.at[pl.ds(base, _CH)], osem[p])
        oc = pltpu.async_copy(cr[p], cc_hbm.at[pl.ds(base, _CH)], osem[p])
        return (ob, oc)

    gathers = {0: load_and_gather(0, 0)}
    outs = {}
    for j in range(n_chunks):
        p = j & 1
        for cp in gathers.pop(j):
            cp.wait()
        if j + 1 < n_chunks:
            if j - 1 in outs:
                for cp in outs.pop(j - 1):
                    cp.wait()
            gathers[j + 1] = load_and_gather(j + 1, p ^ 1)
        outs[j] = store_out(j, p)
    for j in sorted(outs):
        for cp in outs.pop(j):
            cp.wait()


def _edge_body(a_ref, nb_ref, cc_ref, We2_ref, be2_ref, out_ref):
    a = a_ref[0]
    a_lo = a[:, :128]
    a_hi = a[:, 128:]
    nb = nb_ref[0]
    cc = cc_ref[0]
    hw = nb.shape[-1] // _K
    for k in range(_K):
        nb_lo, nb_hi = _unpack_bf16_pair(nb[:, k * hw:(k + 1) * hw])
        cc_lo, cc_hi = _unpack_bf16_pair(cc[:, k * hw:(k + 1) * hw])
        hidden = _gelu(jnp.concatenate(
            [a_lo + nb_lo + cc_lo, a_hi + nb_hi + cc_hi], axis=-1))
        d = 2 * hw
        out_ref[0, :, k * d:(k + 1) * d] = (
            jnp.dot(hidden, We2_ref[...], preferred_element_type=jnp.float32)
            + be2_ref[...])


def kernel(feat_map, W_node, b_node, ln_g, ln_b, W_pos1, b_pos1, W_pos2, b_pos2,
           W_edge1, b_edge1, W_edge2, b_edge2):
    B, C, H, W = feat_map.shape
    N = H * W
    D = W_node.shape[1]
    K = _K
    T = 2304

    fm3 = feat_map.reshape(B, C, N)
    Wa = W_edge1[:D]
    Wb = W_edge1[D:2 * D]
    Wc = W_edge1[2 * D:]
    bn2 = b_node.reshape(1, D)
    g2 = ln_g.reshape(1, D)
    bln2 = ln_b.reshape(1, D)
    be1_2 = b_edge1.reshape(1, D)
    bp1_2 = b_pos1.reshape(1, 64)
    bp2_2 = b_pos2.reshape(1, D)
    be2_2 = b_edge2.reshape(1, D)

    f32 = jnp.float32
    c_table = pl.pallas_call(
        functools.partial(_ctable_body, H),
        grid=(1,),
        in_specs=[
            pl.BlockSpec((2, 64), lambda i: (0, 0)),
            pl.BlockSpec((1, 64), lambda i: (0, 0)),
            pl.BlockSpec((64, D), lambda i: (0, 0)),
            pl.BlockSpec((1, D), lambda i: (0, 0)),
            pl.BlockSpec((D, D), lambda i: (0, 0)),
        ],
        out_specs=pl.BlockSpec((T, D // 2), lambda i: (0, 0)),
        out_shape=jax.ShapeDtypeStruct((T, D // 2), jnp.int32),
    )(W_pos1, bp1_2, W_pos2, bp2_2, Wc)

    nodes, adj, gidx, cidx, a_c, b_c = pl.pallas_call(
        functools.partial(_main_body, H),
        grid=(B,),
        in_specs=[
            pl.BlockSpec((1, C, N), lambda i: (i, 0, 0)),
            pl.BlockSpec((C, D), lambda i: (0, 0)),
            pl.BlockSpec((1, D), lambda i: (0, 0)),
            pl.BlockSpec((1, D), lambda i: (0, 0)),
            pl.BlockSpec((1, D), lambda i: (0, 0)),
            pl.BlockSpec((D, D), lambda i: (0, 0)),
            pl.BlockSpec((D, D), lambda i: (0, 0)),
            pl.BlockSpec((1, D), lambda i: (0, 0)),
        ],
        out_specs=[
            pl.BlockSpec((1, N, D), lambda i: (i, 0, 0)),
            pl.BlockSpec((1, N, K), lambda i: (i, 0, 0)),
            pl.BlockSpec((1, N, K), lambda i: (i, 0, 0)),
            pl.BlockSpec((1, N, K), lambda i: (i, 0, 0)),
            pl.BlockSpec((1, N, D), lambda i: (i, 0, 0)),
            pl.BlockSpec((1, N, D // 2), lambda i: (i, 0, 0)),
        ],
        out_shape=[
            jax.ShapeDtypeStruct((B, N, D), f32),
            jax.ShapeDtypeStruct((B, N, K), jnp.int32),
            jax.ShapeDtypeStruct((B, N, K), jnp.int32),
            jax.ShapeDtypeStruct((B, N, K), jnp.int32),
            jax.ShapeDtypeStruct((B, N, D), f32),
            jax.ShapeDtypeStruct((B, N, D // 2), jnp.int32),
        ],
    )(fm3, W_node, bn2, g2, bln2, Wa, Wb, be1_2)

    total = B * N * K
    n_chunks = total // (_NW * _CH)
    mesh = plsc.VectorSubcoreMesh(core_axis_name="c", subcore_axis_name="s")
    nb, cc = pl.kernel(
        functools.partial(_gather_body, n_chunks),
        mesh=mesh,
        out_type=[
            jax.ShapeDtypeStruct((total, D // 2), jnp.int32),
            jax.ShapeDtypeStruct((total, D // 2), jnp.int32),
        ],
        scratch_types=[
            pltpu.VMEM((_CH,), jnp.int32),
            pltpu.VMEM((_CH,), jnp.int32),
            pltpu.VMEM((_CH,), jnp.int32),
            pltpu.VMEM((_CH,), jnp.int32),
            pltpu.VMEM((_CH, D // 2), jnp.int32),
            pltpu.VMEM((_CH, D // 2), jnp.int32),
            pltpu.VMEM((_CH, D // 2), jnp.int32),
            pltpu.VMEM((_CH, D // 2), jnp.int32),
            pltpu.SemaphoreType.DMA,
            pltpu.SemaphoreType.DMA,
            pltpu.SemaphoreType.DMA,
            pltpu.SemaphoreType.DMA,
        ],
    )(b_c.reshape(B * N, D // 2), c_table, gidx.reshape(total), cidx.reshape(total))

    edges = pl.pallas_call(
        _edge_body,
        grid=(B,),
        in_specs=[
            pl.BlockSpec((1, N, D), lambda i: (i, 0, 0)),
            pl.BlockSpec((1, N, K * D // 2), lambda i: (i, 0, 0)),
            pl.BlockSpec((1, N, K * D // 2), lambda i: (i, 0, 0)),
            pl.BlockSpec((D, D), lambda i: (0, 0)),
            pl.BlockSpec((1, D), lambda i: (0, 0)),
        ],
        out_specs=pl.BlockSpec((1, N, K * D), lambda i: (i, 0, 0)),
        out_shape=jax.ShapeDtypeStruct((B, N, K * D), f32),
    )(a_c, nb.reshape(B, N, K * D // 2), cc.reshape(B, N, K * D // 2),
      W_edge2, be2_2)

    return (nodes, edges.reshape(B, N, K, D), adj)


# speedup vs baseline: 1.2173x; 1.2173x over previous
import functools

import jax
import jax.numpy as jnp
from jax import lax
from jax.experimental import pallas as pl
from jax.experimental.pallas import tpu as pltpu
from jax.experimental.pallas import tpu_sc as plsc

_K = 8
_SQRT_HALF = 0.7071067811865476
_NC = 2
_NS = 16
_NW = _NC * _NS
_CH = 128


def _gelu(x):
    return 0.5 * x * (1.0 + lax.erf(x * jnp.float32(_SQRT_HALF)))


def _pack_bf16_pair(x):
    r = lax.bitcast_convert_type(x, jnp.uint32)
    r16 = (r + jnp.uint32(0x7FFF) + ((r >> 16) & jnp.uint32(1))) >> 16
    lo = r16[:, :128]
    hi = r16[:, 128:]
    return lax.bitcast_convert_type(lo | (hi << 16), jnp.int32)


def _unpack_bf16_pair(p):
    u = lax.bitcast_convert_type(p, jnp.uint32)
    f_lo = lax.bitcast_convert_type(u << 16, jnp.float32)
    f_hi = lax.bitcast_convert_type(u & jnp.uint32(0xFFFF0000), jnp.float32)
    return f_lo, f_hi


def _ctable_body(h_grid, Wp1_ref, bp1_ref, Wp2_ref, bp2_ref, Wc_ref, out_ref):
    T = out_ref.shape[0]
    S = 2 * h_grid - 1
    step = jnp.float32(1.0 / (h_grid - 1))
    Wc_comb = jnp.dot(Wp2_ref[...], Wc_ref[...], preferred_element_type=jnp.float32)
    bc_comb = jnp.dot(bp2_ref[...], Wc_ref[...], preferred_element_type=jnp.float32)
    d = lax.broadcasted_iota(jnp.int32, (T, 1), 0)
    dy = (d // S - (h_grid - 1)).astype(jnp.float32) * step
    dx = (d % S - (h_grid - 1)).astype(jnp.float32) * step
    g1 = _gelu(dy * Wp1_ref[0:1, :] + dx * Wp1_ref[1:2, :] + bp1_ref[...])
    ct = jnp.dot(g1, Wc_comb, preferred_element_type=jnp.float32) + bc_comb
    out_ref[...] = _pack_bf16_pair(ct)


def _main_body(h_grid, fm_ref, Wn_ref, bn_ref, g_ref, bln_ref, Wa_ref, Wb_ref,
               be1_ref, nodes_ref, adj_ref, gidx_ref, cidx_ref, a_ref, b_ref):
    N = fm_ref.shape[2]
    fm = fm_ref[0]
    h = lax.dot_general(fm, Wn_ref[...], (((0,), (0,)), ((), ())),
                        preferred_element_type=jnp.float32) + bn_ref[...]
    mu = jnp.mean(h, axis=-1, keepdims=True)
    var = jnp.mean((h - mu) ** 2, axis=-1, keepdims=True)
    h = (h - mu) / jnp.sqrt(var + 1e-5) * g_ref[...] + bln_ref[...]
    nodes = _gelu(h)
    nodes_ref[0] = nodes
    ss = jnp.sum(nodes * nodes, axis=-1, keepdims=True)
    norm = jnp.maximum(jnp.sqrt(ss), 1e-12)
    nrm = nodes / norm
    sim = lax.dot_general(nrm, nrm, (((1,), (1,)), ((), ())),
                          preferred_element_type=jnp.float32)
    rid = lax.broadcasted_iota(jnp.int32, (N, N), 0)
    cid = lax.broadcasted_iota(jnp.int32, (N, N), 1)
    sim = jnp.where(rid == cid, sim - 1e9, sim)
    cols = []
    s = sim
    for _ in range(_K):
        vmax = jnp.max(s, axis=-1, keepdims=True)
        idx = jnp.min(jnp.where(s == vmax, cid, N), axis=-1, keepdims=True)
        cols.append(idx)
        s = jnp.where(cid == idx, -jnp.inf, s)
    adj = jnp.concatenate(cols, axis=-1)
    adj_ref[0] = adj
    gidx_ref[0] = adj + pl.program_id(0) * N
    S = 2 * h_grid - 1
    rown = lax.broadcasted_iota(jnp.int32, (N, _K), 0)
    dyi = adj // h_grid - rown // h_grid + (h_grid - 1)
    dxi = adj % h_grid - rown % h_grid + (h_grid - 1)
    cidx_ref[0] = dyi * S + dxi
    a_ref[0] = _pack_bf16_pair(
        jnp.dot(nodes, Wa_ref[...], preferred_element_type=jnp.float32)
        + be1_ref[...])
    b_ref[0] = _pack_bf16_pair(
        jnp.dot(nodes, Wb_ref[...], preferred_element_type=jnp.float32))


def _gather_body(n_chunks, atab_hbm, btab_hbm, ctab_hbm, aidx_hbm, bidx_hbm,
                 cidx_hbm, aa_hbm, nb_hbm, cc_hbm,
                 ai0, ai1, bi0, bi1, ci0, ci1,
                 ar0, ar1, br0, br1, cr0, cr1,
                 gsem0, gsem1, osem0, osem1):
    wid = lax.axis_index("s") * _NC + lax.axis_index("c")
    span = n_chunks * _CH
    ai = (ai0, ai1)
    bi = (bi0, bi1)
    ci = (ci0, ci1)
    ar = (ar0, ar1)
    br = (br0, br1)
    cr = (cr0, cr1)
    gsem = (gsem0, gsem1)
    osem = (osem0, osem1)

    def load_and_gather(j, p):
        base = wid * span + j * _CH
        pltpu.sync_copy(aidx_hbm.at[pl.ds(base, _CH)], ai[p])
        pltpu.sync_copy(bidx_hbm.at[pl.ds(base, _CH)], bi[p])
        pltpu.sync_copy(cidx_hbm.at[pl.ds(base, _CH)], ci[p])
        ga = pltpu.async_copy(atab_hbm.at[ai[p]], ar[p], gsem[p])
        gb = pltpu.async_copy(btab_hbm.at[bi[p]], br[p], gsem[p])
        gc = pltpu.async_copy(ctab_hbm.at[ci[p]], cr[p], gsem[p])
        return (ga, gb, gc)

    def store_out(j, p):
        base = wid * span + j * _CH
        oa = pltpu.async_copy(ar[p], aa_hbm.at[pl.ds(base, _CH)], osem[p])
        ob = pltpu.async_copy(br[p], nb_hbm.at[pl.ds(base, _CH)], osem[p])
        oc = pltpu.async_copy(cr[p], cc_hbm.at[pl.ds(base, _CH)], osem[p])
        return (oa, ob, oc)

    gathers = {0: load_and_gather(0, 0)}
    outs = {}
    for j in range(n_chunks):
        p = j & 1
        for cp in gathers.pop(j):
            cp.wait()
        if j + 1 < n_chunks:
            if j - 1 in outs:
                for cp in outs.pop(j - 1):
                    cp.wait()
            gathers[j + 1] = load_and_gather(j + 1, p ^ 1)
        outs[j] = store_out(j, p)
    for j in sorted(outs):
        for cp in outs.pop(j):
            cp.wait()


def _edge_body(aa_ref, nb_ref, cc_ref, We2_ref, be2_ref, out_ref):
    a_lo, a_hi = _unpack_bf16_pair(aa_ref[0])
    nb_lo, nb_hi = _unpack_bf16_pair(nb_ref[0])
    cc_lo, cc_hi = _unpack_bf16_pair(cc_ref[0])
    hidden = _gelu(jnp.concatenate(
        [a_lo + nb_lo + cc_lo, a_hi + nb_hi + cc_hi], axis=-1))
    out_ref[0] = (
        jnp.dot(hidden, We2_ref[...], preferred_element_type=jnp.float32)
        + be2_ref[...])


def kernel(feat_map, W_node, b_node, ln_g, ln_b, W_pos1, b_pos1, W_pos2, b_pos2,
           W_edge1, b_edge1, W_edge2, b_edge2):
    B, C, H, W = feat_map.shape
    N = H * W
    D = W_node.shape[1]
    K = _K
    T = 2304

    fm3 = feat_map.reshape(B, C, N)
    Wa = W_edge1[:D]
    Wb = W_edge1[D:2 * D]
    Wc = W_edge1[2 * D:]
    bn2 = b_node.reshape(1, D)
    g2 = ln_g.reshape(1, D)
    bln2 = ln_b.reshape(1, D)
    be1_2 = b_edge1.reshape(1, D)
    bp1_2 = b_pos1.reshape(1, 64)
    bp2_2 = b_pos2.reshape(1, D)
    be2_2 = b_edge2.reshape(1, D)

    f32 = jnp.float32
    c_table = pl.pallas_call(
        functools.partial(_ctable_body, H),
        grid=(1,),
        in_specs=[
            pl.BlockSpec((2, 64), lambda i: (0, 0)),
            pl.BlockSpec((1, 64), lambda i: (0, 0)),
            pl.BlockSpec((64, D), lambda i: (0, 0)),
            pl.BlockSpec((1, D), lambda i: (0, 0)),
            pl.BlockSpec((D, D), lambda i: (0, 0)),
        ],
        out_specs=pl.BlockSpec((T, D // 2), lambda i: (0, 0)),
        out_shape=jax.ShapeDtypeStruct((T, D // 2), jnp.int32),
    )(W_pos1, bp1_2, W_pos2, bp2_2, Wc)

    nodes, adj, gidx, cidx, a_c, b_c = pl.pallas_call(
        functools.partial(_main_body, H),
        grid=(B,),
        in_specs=[
            pl.BlockSpec((1, C, N), lambda i: (i, 0, 0)),
            pl.BlockSpec((C, D), lambda i: (0, 0)),
            pl.BlockSpec((1, D), lambda i: (0, 0)),
            pl.BlockSpec((1, D), lambda i: (0, 0)),
            pl.BlockSpec((1, D), lambda i: (0, 0)),
            pl.BlockSpec((D, D), lambda i: (0, 0)),
            pl.BlockSpec((D, D), lambda i: (0, 0)),
            pl.BlockSpec((1, D), lambda i: (0, 0)),
        ],
        out_specs=[
            pl.BlockSpec((1, N, D), lambda i: (i, 0, 0)),
            pl.BlockSpec((1, N, K), lambda i: (i, 0, 0)),
            pl.BlockSpec((1, N, K), lambda i: (i, 0, 0)),
            pl.BlockSpec((1, N, K), lambda i: (i, 0, 0)),
            pl.BlockSpec((1, N, D // 2), lambda i: (i, 0, 0)),
            pl.BlockSpec((1, N, D // 2), lambda i: (i, 0, 0)),
        ],
        out_shape=[
            jax.ShapeDtypeStruct((B, N, D), f32),
            jax.ShapeDtypeStruct((B, N, K), jnp.int32),
            jax.ShapeDtypeStruct((B, N, K), jnp.int32),
            jax.ShapeDtypeStruct((B, N, K), jnp.int32),
            jax.ShapeDtypeStruct((B, N, D // 2), jnp.int32),
            jax.ShapeDtypeStruct((B, N, D // 2), jnp.int32),
        ],
    )(fm3, W_node, bn2, g2, bln2, Wa, Wb, be1_2)

    total = B * N * K
    n_chunks = total // (_NW * _CH)
    aidx = jnp.arange(total, dtype=jnp.int32) // K
    mesh = plsc.VectorSubcoreMesh(core_axis_name="c", subcore_axis_name="s")
    aa, nb, cc = pl.kernel(
        functools.partial(_gather_body, n_chunks),
        mesh=mesh,
        out_type=[
            jax.ShapeDtypeStruct((total, D // 2), jnp.int32),
            jax.ShapeDtypeStruct((total, D // 2), jnp.int32),
            jax.ShapeDtypeStruct((total, D // 2), jnp.int32),
        ],
        scratch_types=(
            [pltpu.VMEM((_CH,), jnp.int32)] * 6
            + [pltpu.VMEM((_CH, D // 2), jnp.int32)] * 6
            + [pltpu.SemaphoreType.DMA] * 4
        ),
    )(a_c.reshape(B * N, D // 2), b_c.reshape(B * N, D // 2), c_table,
      aidx, gidx.reshape(total), cidx.reshape(total))

    edges = pl.pallas_call(
        _edge_body,
        grid=(B,),
        in_specs=[
            pl.BlockSpec((1, N * K, D // 2), lambda i: (i, 0, 0)),
            pl.BlockSpec((1, N * K, D // 2), lambda i: (i, 0, 0)),
            pl.BlockSpec((1, N * K, D // 2), lambda i: (i, 0, 0)),
            pl.BlockSpec((D, D), lambda i: (0, 0)),
            pl.BlockSpec((1, D), lambda i: (0, 0)),
        ],
        out_specs=pl.BlockSpec((1, N * K, D), lambda i: (i, 0, 0)),
        out_shape=jax.ShapeDtypeStruct((B, N * K, D), f32),
    )(aa.reshape(B, N * K, D // 2), nb.reshape(B, N * K, D // 2),
      cc.reshape(B, N * K, D // 2), W_edge2, be2_2)

    return (nodes, edges.reshape(B, N, K, D), adj)


# speedup vs baseline: 1.4663x; 1.2046x over previous
import functools

import jax
import jax.numpy as jnp
from jax import lax
from jax.experimental import pallas as pl
from jax.experimental.pallas import tpu as pltpu
from jax.experimental.pallas import tpu_sc as plsc

_K = 8
_SQRT_HALF = 0.7071067811865476
_NC = 2
_NS = 16
_NW = _NC * _NS
_CH = 128


def _gelu(x):
    return 0.5 * x * (1.0 + lax.erf(x * jnp.float32(_SQRT_HALF)))


def _pack_bf16_pair(x):
    r = lax.bitcast_convert_type(x, jnp.uint32)
    r16 = (r + jnp.uint32(0x7FFF) + ((r >> 16) & jnp.uint32(1))) >> 16
    lo = r16[:, :128]
    hi = r16[:, 128:]
    return lax.bitcast_convert_type(lo | (hi << 16), jnp.int32)


def _unpack_bf16_pair(p):
    u = lax.bitcast_convert_type(p, jnp.uint32)
    f_lo = lax.bitcast_convert_type(u << 16, jnp.float32)
    f_hi = lax.bitcast_convert_type(u & jnp.uint32(0xFFFF0000), jnp.float32)
    return f_lo, f_hi


def _ctable(h_grid, T, Wp1_ref, bp1_ref, Wp2_ref, bp2_ref, Wc_ref):
    S = 2 * h_grid - 1
    step = jnp.float32(1.0 / (h_grid - 1))
    Wc_comb = jnp.dot(Wp2_ref[...], Wc_ref[...], preferred_element_type=jnp.float32)
    bc_comb = jnp.dot(bp2_ref[...], Wc_ref[...], preferred_element_type=jnp.float32)
    d = lax.broadcasted_iota(jnp.int32, (T, 1), 0)
    dy = (d // S - (h_grid - 1)).astype(jnp.float32) * step
    dx = (d % S - (h_grid - 1)).astype(jnp.float32) * step
    g1 = _gelu(dy * Wp1_ref[0:1, :] + dx * Wp1_ref[1:2, :] + bp1_ref[...])
    ct = jnp.dot(g1, Wc_comb, preferred_element_type=jnp.float32) + bc_comb
    return _pack_bf16_pair(ct)


def _main_body(h_grid, fm_ref, Wn_ref, bn_ref, g_ref, bln_ref, Wa_ref, Wb_ref,
               be1_ref, Wp1_ref, bp1_ref, Wp2_ref, bp2_ref, Wc_ref,
               nodes_ref, adj_ref, gidx_ref, cidx_ref, a_ref, b_ref, ct_ref):
    @pl.when(pl.program_id(0) == 0)
    def _():
        ct_ref[...] = _ctable(h_grid, ct_ref.shape[0], Wp1_ref, bp1_ref,
                              Wp2_ref, bp2_ref, Wc_ref)
    N = fm_ref.shape[2]
    fm = fm_ref[0]
    h = lax.dot_general(fm, Wn_ref[...], (((0,), (0,)), ((), ())),
                        preferred_element_type=jnp.float32) + bn_ref[...]
    mu = jnp.mean(h, axis=-1, keepdims=True)
    var = jnp.mean((h - mu) ** 2, axis=-1, keepdims=True)
    h = (h - mu) / jnp.sqrt(var + 1e-5) * g_ref[...] + bln_ref[...]
    nodes = _gelu(h)
    nodes_ref[0] = nodes
    ss = jnp.sum(nodes * nodes, axis=-1, keepdims=True)
    norm = jnp.maximum(jnp.sqrt(ss), 1e-12)
    nrm = nodes / norm
    sim = lax.dot_general(nrm, nrm, (((1,), (1,)), ((), ())),
                          preferred_element_type=jnp.float32)
    rid = lax.broadcasted_iota(jnp.int32, (N, N), 0)
    cid = lax.broadcasted_iota(jnp.int32, (N, N), 1)
    sim = jnp.where(rid == cid, sim - 1e9, sim)
    cols = []
    s = sim
    for _ in range(_K):
        vmax = jnp.max(s, axis=-1, keepdims=True)
        idx = jnp.min(jnp.where(s == vmax, cid, N), axis=-1, keepdims=True)
        cols.append(idx)
        s = jnp.where(cid == idx, -jnp.inf, s)
    adj = jnp.concatenate(cols, axis=-1)
    adj_ref[0] = adj
    gidx_ref[0] = adj + pl.program_id(0) * N
    S = 2 * h_grid - 1
    rown = lax.broadcasted_iota(jnp.int32, (N, _K), 0)
    dyi = adj // h_grid - rown // h_grid + (h_grid - 1)
    dxi = adj % h_grid - rown % h_grid + (h_grid - 1)
    cidx_ref[0] = dyi * S + dxi
    a_ref[0] = (jnp.dot(nodes, Wa_ref[...], preferred_element_type=jnp.float32)
                + be1_ref[...])
    b_ref[0] = _pack_bf16_pair(
        jnp.dot(nodes, Wb_ref[...], preferred_element_type=jnp.float32))


def _gather_body(n_chunks, btab_hbm, ctab_hbm, bidx_hbm, cidx_hbm,
                 nb_hbm, cc_hbm, bi0, bi1, ci0, ci1, br0, br1, cr0, cr1,
                 gsem0, gsem1, osem0, osem1):
    wid = lax.axis_index("s") * _NC + lax.axis_index("c")
    span = n_chunks * _CH
    bi = (bi0, bi1)
    ci = (ci0, ci1)
    br = (br0, br1)
    cr = (cr0, cr1)
    gsem = (gsem0, gsem1)
    osem = (osem0, osem1)

    def load_and_gather(j, p):
        base = wid * span + j * _CH
        pltpu.sync_copy(bidx_hbm.at[pl.ds(base, _CH)], bi[p])
        pltpu.sync_copy(cidx_hbm.at[pl.ds(base, _CH)], ci[p])
        gb = pltpu.async_copy(btab_hbm.at[bi[p]], br[p], gsem[p])
        gc = pltpu.async_copy(ctab_hbm.at[ci[p]], cr[p], gsem[p])
        return (gb, gc)

    def store_out(j, p):
        base = wid * span + j * _CH
        ob = pltpu.async_copy(br[p], nb_hbm.at[pl.ds(base, _CH)], osem[p])
        oc = pltpu.async_copy(cr[p], cc_hbm.at[pl.ds(base, _CH)], osem[p])
        return (ob, oc)

    gathers = {0: load_and_gather(0, 0)}
    outs = {}
    for j in range(n_chunks):
        p = j & 1
        for cp in gathers.pop(j):
            cp.wait()
        if j + 1 < n_chunks:
            if j - 1 in outs:
                for cp in outs.pop(j - 1):
                    cp.wait()
            gathers[j + 1] = load_and_gather(j + 1, p ^ 1)
        outs[j] = store_out(j, p)
    for j in sorted(outs):
        for cp in outs.pop(j):
            cp.wait()


def _edge_body(a_ref, nb_ref, cc_ref, We2_ref, be2_ref, out_ref):
    NK = nb_ref.shape[1]
    N = NK // _K
    hw = nb_ref.shape[2]
    a = a_ref[0]
    nb_lo, nb_hi = _unpack_bf16_pair(nb_ref[0])
    cc_lo, cc_hi = _unpack_bf16_pair(cc_ref[0])
    s_lo = (nb_lo + cc_lo).reshape(N, _K, hw) + a[:, None, :hw]
    s_hi = (nb_hi + cc_hi).reshape(N, _K, hw) + a[:, None, hw:]
    hidden = _gelu(jnp.concatenate([s_lo, s_hi], axis=-1)).reshape(NK, 2 * hw)
    out_ref[0] = (
        jnp.dot(hidden, We2_ref[...], preferred_element_type=jnp.float32)
        + be2_ref[...])


def kernel(feat_map, W_node, b_node, ln_g, ln_b, W_pos1, b_pos1, W_pos2, b_pos2,
           W_edge1, b_edge1, W_edge2, b_edge2):
    B, C, H, W = feat_map.shape
    N = H * W
    D = W_node.shape[1]
    K = _K
    T = 2304

    fm3 = feat_map.reshape(B, C, N)
    Wa = W_edge1[:D]
    Wb = W_edge1[D:2 * D]
    Wc = W_edge1[2 * D:]
    bn2 = b_node.reshape(1, D)
    g2 = ln_g.reshape(1, D)
    bln2 = ln_b.reshape(1, D)
    be1_2 = b_edge1.reshape(1, D)
    bp1_2 = b_pos1.reshape(1, 64)
    bp2_2 = b_pos2.reshape(1, D)
    be2_2 = b_edge2.reshape(1, D)

    f32 = jnp.float32
    nodes, adj, gidx, cidx, a_c, b_c, c_table = pl.pallas_call(
        functools.partial(_main_body, H),
        grid=(B,),
        in_specs=[
            pl.BlockSpec((1, C, N), lambda i: (i, 0, 0)),
            pl.BlockSpec((C, D), lambda i: (0, 0)),
            pl.BlockSpec((1, D), lambda i: (0, 0)),
            pl.BlockSpec((1, D), lambda i: (0, 0)),
            pl.BlockSpec((1, D), lambda i: (0, 0)),
            pl.BlockSpec((D, D), lambda i: (0, 0)),
            pl.BlockSpec((D, D), lambda i: (0, 0)),
            pl.BlockSpec((1, D), lambda i: (0, 0)),
            pl.BlockSpec((2, 64), lambda i: (0, 0)),
            pl.BlockSpec((1, 64), lambda i: (0, 0)),
            pl.BlockSpec((64, D), lambda i: (0, 0)),
            pl.BlockSpec((1, D), lambda i: (0, 0)),
            pl.BlockSpec((D, D), lambda i: (0, 0)),
        ],
        out_specs=[
            pl.BlockSpec((1, N, D), lambda i: (i, 0, 0)),
            pl.BlockSpec((1, N, K), lambda i: (i, 0, 0)),
            pl.BlockSpec((1, N, K), lambda i: (i, 0, 0)),
            pl.BlockSpec((1, N, K), lambda i: (i, 0, 0)),
            pl.BlockSpec((1, N, D), lambda i: (i, 0, 0)),
            pl.BlockSpec((1, N, D // 2), lambda i: (i, 0, 0)),
            pl.BlockSpec((T, D // 2), lambda i: (0, 0)),
        ],
        out_shape=[
            jax.ShapeDtypeStruct((B, N, D), f32),
            jax.ShapeDtypeStruct((B, N, K), jnp.int32),
            jax.ShapeDtypeStruct((B, N, K), jnp.int32),
            jax.ShapeDtypeStruct((B, N, K), jnp.int32),
            jax.ShapeDtypeStruct((B, N, D), f32),
            jax.ShapeDtypeStruct((B, N, D // 2), jnp.int32),
            jax.ShapeDtypeStruct((T, D // 2), jnp.int32),
        ],
    )(fm3, W_node, bn2, g2, bln2, Wa, Wb, be1_2,
      W_pos1, bp1_2, W_pos2, bp2_2, Wc)

    total = B * N * K
    n_chunks = total // (_NW * _CH)
    mesh = plsc.VectorSubcoreMesh(core_axis_name="c", subcore_axis_name="s")
    nb, cc = pl.kernel(
        functools.partial(_gather_body, n_chunks),
        mesh=mesh,
        out_type=[
            jax.ShapeDtypeStruct((total, D // 2), jnp.int32),
            jax.ShapeDtypeStruct((total, D // 2), jnp.int32),
        ],
        scratch_types=(
            [pltpu.VMEM((_CH,), jnp.int32)] * 4
            + [pltpu.VMEM((_CH, D // 2), jnp.int32)] * 4
            + [pltpu.SemaphoreType.DMA] * 4
        ),
    )(b_c.reshape(B * N, D // 2), c_table,
      gidx.reshape(total), cidx.reshape(total))

    edges = pl.pallas_call(
        _edge_body,
        grid=(B,),
        in_specs=[
            pl.BlockSpec((1, N, D), lambda i: (i, 0, 0)),
            pl.BlockSpec((1, N * K, D // 2), lambda i: (i, 0, 0)),
            pl.BlockSpec((1, N * K, D // 2), lambda i: (i, 0, 0)),
            pl.BlockSpec((D, D), lambda i: (0, 0)),
            pl.BlockSpec((1, D), lambda i: (0, 0)),
        ],
        out_specs=pl.BlockSpec((1, N * K, D), lambda i: (i, 0, 0)),
        out_shape=jax.ShapeDtypeStruct((B, N * K, D), f32),
    )(a_c, nb.reshape(B, N * K, D // 2),
      cc.reshape(B, N * K, D // 2), W_edge2, be2_2)

    return (nodes, edges.reshape(B, N, K, D), adj)


# speedup vs baseline: 1.5135x; 1.0322x over previous
import functools

import jax
import jax.numpy as jnp
from jax import lax
from jax.experimental import pallas as pl
from jax.experimental.pallas import tpu as pltpu
from jax.experimental.pallas import tpu_sc as plsc

_K = 8
_SQRT_HALF = 0.7071067811865476
_NC = 2
_NS = 16
_NW = _NC * _NS
_CH = 128


def _gelu(x):
    return 0.5 * x * (1.0 + lax.erf(x * jnp.float32(_SQRT_HALF)))


def _pack_bf16_pair(x):
    r = lax.bitcast_convert_type(x, jnp.uint32)
    r16 = (r + jnp.uint32(0x7FFF) + ((r >> 16) & jnp.uint32(1))) >> 16
    lo = r16[:, :128]
    hi = r16[:, 128:]
    return lax.bitcast_convert_type(lo | (hi << 16), jnp.int32)


def _unpack_bf16_pair(p):
    u = lax.bitcast_convert_type(p, jnp.uint32)
    f_lo = lax.bitcast_convert_type(u << 16, jnp.float32)
    f_hi = lax.bitcast_convert_type(u & jnp.uint32(0xFFFF0000), jnp.float32)
    return f_lo, f_hi


def _ctable(h_grid, T, Wp1_ref, bp1_ref, Wp2_ref, bp2_ref, Wc_ref):
    S = 2 * h_grid - 1
    step = jnp.float32(1.0 / (h_grid - 1))
    Wc_comb = jnp.dot(Wp2_ref[...], Wc_ref[...], preferred_element_type=jnp.float32)
    bc_comb = jnp.dot(bp2_ref[...], Wc_ref[...], preferred_element_type=jnp.float32)
    d = lax.broadcasted_iota(jnp.int32, (T, 1), 0)
    dy = (d // S - (h_grid - 1)).astype(jnp.float32) * step
    dx = (d % S - (h_grid - 1)).astype(jnp.float32) * step
    g1 = _gelu(dy * Wp1_ref[0:1, :] + dx * Wp1_ref[1:2, :] + bp1_ref[...])
    ct = jnp.dot(g1, Wc_comb, preferred_element_type=jnp.float32) + bc_comb
    return _pack_bf16_pair(ct)


def _main_body(h_grid, fm_ref, Wn_ref, bn_ref, g_ref, bln_ref, Wa_ref, Wb_ref,
               be1_ref, Wp1_ref, bp1_ref, Wp2_ref, bp2_ref, Wc_ref,
               nodes_ref, adj_ref, gidx_ref, cidx_ref, a_ref, b_ref, ct_ref):
    @pl.when(pl.program_id(0) == 0)
    def _():
        ct_ref[...] = _ctable(h_grid, ct_ref.shape[0], Wp1_ref, bp1_ref,
                              Wp2_ref, bp2_ref, Wc_ref)
    N = fm_ref.shape[2]
    fm = fm_ref[0]
    h = lax.dot_general(fm, Wn_ref[...], (((0,), (0,)), ((), ())),
                        preferred_element_type=jnp.float32) + bn_ref[...]
    mu = jnp.mean(h, axis=-1, keepdims=True)
    var = jnp.mean((h - mu) ** 2, axis=-1, keepdims=True)
    h = (h - mu) / jnp.sqrt(var + 1e-5) * g_ref[...] + bln_ref[...]
    nodes = _gelu(h)
    nodes_ref[0] = nodes
    ss = jnp.sum(nodes * nodes, axis=-1, keepdims=True)
    norm = jnp.maximum(jnp.sqrt(ss), 1e-12)
    nrm = nodes / norm
    sim = lax.dot_general(nrm, nrm, (((1,), (1,)), ((), ())),
                          preferred_element_type=jnp.float32)
    rid = lax.broadcasted_iota(jnp.int32, (N, N), 0)
    cid = lax.broadcasted_iota(jnp.int32, (N, N), 1)
    sim = jnp.where(rid == cid, sim - 1e9, sim)
    cols = []
    s = sim
    for _ in range(_K):
        vmax = jnp.max(s, axis=-1, keepdims=True)
        idx = jnp.min(jnp.where(s == vmax, cid, N), axis=-1, keepdims=True)
        cols.append(idx)
        s = jnp.where(cid == idx, -jnp.inf, s)
    adj = jnp.concatenate(cols, axis=-1)
    adj_ref[0] = adj
    gidx_ref[0] = adj + pl.program_id(0) * N
    S = 2 * h_grid - 1
    rown = lax.broadcasted_iota(jnp.int32, (N, _K), 0)
    dyi = adj // h_grid - rown // h_grid + (h_grid - 1)
    dxi = adj % h_grid - rown % h_grid + (h_grid - 1)
    cidx_ref[0] = dyi * S + dxi
    a_ref[0] = (jnp.dot(nodes, Wa_ref[...], preferred_element_type=jnp.float32)
                + be1_ref[...])
    b_ref[0] = _pack_bf16_pair(
        jnp.dot(nodes, Wb_ref[...], preferred_element_type=jnp.float32))


def _gather_body(n_chunks, btab_hbm, ctab_hbm, bidx_hbm, cidx_hbm,
                 nb_hbm, cc_hbm, bi_all, ci_all, br0, br1, cr0, cr1,
                 gsem0, gsem1, osem0, osem1):
    wid = lax.axis_index("s") * _NC + lax.axis_index("c")
    span = n_chunks * _CH
    br = (br0, br1)
    cr = (cr0, cr1)
    gsem = (gsem0, gsem1)
    osem = (osem0, osem1)

    pltpu.sync_copy(bidx_hbm.at[pl.ds(wid * span, span)], bi_all)
    pltpu.sync_copy(cidx_hbm.at[pl.ds(wid * span, span)], ci_all)

    def load_and_gather(j, p):
        gb = pltpu.async_copy(
            btab_hbm.at[bi_all.at[pl.ds(j * _CH, _CH)]], br[p], gsem[p])
        gc = pltpu.async_copy(
            ctab_hbm.at[ci_all.at[pl.ds(j * _CH, _CH)]], cr[p], gsem[p])
        return (gb, gc)

    def store_out(j, p):
        base = wid * span + j * _CH
        ob = pltpu.async_copy(br[p], nb_hbm.at[pl.ds(base, _CH)], osem[p])
        oc = pltpu.async_copy(cr[p], cc_hbm.at[pl.ds(base, _CH)], osem[p])
        return (ob, oc)

    gathers = {0: load_and_gather(0, 0)}
    outs = {}
    for j in range(n_chunks):
        p = j & 1
        for cp in gathers.pop(j):
            cp.wait()
        if j + 1 < n_chunks:
            if j - 1 in outs:
                for cp in outs.pop(j - 1):
                    cp.wait()
            gathers[j + 1] = load_and_gather(j + 1, p ^ 1)
        outs[j] = store_out(j, p)
    for j in sorted(outs):
        for cp in outs.pop(j):
            cp.wait()


def _edge_body(a_ref, nb_ref, cc_ref, We2_ref, be2_ref, out_ref):
    NK = nb_ref.shape[1]
    N = NK // _K
    hw = nb_ref.shape[2]
    a = a_ref[0]
    nb_lo, nb_hi = _unpack_bf16_pair(nb_ref[0])
    cc_lo, cc_hi = _unpack_bf16_pair(cc_ref[0])
    s_lo = (nb_lo + cc_lo).reshape(N, _K, hw) + a[:, None, :hw]
    s_hi = (nb_hi + cc_hi).reshape(N, _K, hw) + a[:, None, hw:]
    hidden = _gelu(jnp.concatenate([s_lo, s_hi], axis=-1)).reshape(NK, 2 * hw)
    out_ref[0] = (
        jnp.dot(hidden, We2_ref[...], preferred_element_type=jnp.float32)
        + be2_ref[...])


def kernel(feat_map, W_node, b_node, ln_g, ln_b, W_pos1, b_pos1, W_pos2, b_pos2,
           W_edge1, b_edge1, W_edge2, b_edge2):
    B, C, H, W = feat_map.shape
    N = H * W
    D = W_node.shape[1]
    K = _K
    T = 2304

    fm3 = feat_map.reshape(B, C, N)
    Wa = W_edge1[:D]
    Wb = W_edge1[D:2 * D]
    Wc = W_edge1[2 * D:]
    bn2 = b_node.reshape(1, D)
    g2 = ln_g.reshape(1, D)
    bln2 = ln_b.reshape(1, D)
    be1_2 = b_edge1.reshape(1, D)
    bp1_2 = b_pos1.reshape(1, 64)
    bp2_2 = b_pos2.reshape(1, D)
    be2_2 = b_edge2.reshape(1, D)

    f32 = jnp.float32
    nodes, adj, gidx, cidx, a_c, b_c, c_table = pl.pallas_call(
        functools.partial(_main_body, H),
        grid=(B,),
        in_specs=[
            pl.BlockSpec((1, C, N), lambda i: (i, 0, 0)),
            pl.BlockSpec((C, D), lambda i: (0, 0)),
            pl.BlockSpec((1, D), lambda i: (0, 0)),
            pl.BlockSpec((1, D), lambda i: (0, 0)),
            pl.BlockSpec((1, D), lambda i: (0, 0)),
            pl.BlockSpec((D, D), lambda i: (0, 0)),
            pl.BlockSpec((D, D), lambda i: (0, 0)),
            pl.BlockSpec((1, D), lambda i: (0, 0)),
            pl.BlockSpec((2, 64), lambda i: (0, 0)),
            pl.BlockSpec((1, 64), lambda i: (0, 0)),
            pl.BlockSpec((64, D), lambda i: (0, 0)),
            pl.BlockSpec((1, D), lambda i: (0, 0)),
            pl.BlockSpec((D, D), lambda i: (0, 0)),
        ],
        out_specs=[
            pl.BlockSpec((1, N, D), lambda i: (i, 0, 0)),
            pl.BlockSpec((1, N, K), lambda i: (i, 0, 0)),
            pl.BlockSpec((1, N, K), lambda i: (i, 0, 0)),
            pl.BlockSpec((1, N, K), lambda i: (i, 0, 0)),
            pl.BlockSpec((1, N, D), lambda i: (i, 0, 0)),
            pl.BlockSpec((1, N, D // 2), lambda i: (i, 0, 0)),
            pl.BlockSpec((T, D // 2), lambda i: (0, 0)),
        ],
        out_shape=[
            jax.ShapeDtypeStruct((B, N, D), f32),
            jax.ShapeDtypeStruct((B, N, K), jnp.int32),
            jax.ShapeDtypeStruct((B, N, K), jnp.int32),
            jax.ShapeDtypeStruct((B, N, K), jnp.int32),
            jax.ShapeDtypeStruct((B, N, D), f32),
            jax.ShapeDtypeStruct((B, N, D // 2), jnp.int32),
            jax.ShapeDtypeStruct((T, D // 2), jnp.int32),
        ],
    )(fm3, W_node, bn2, g2, bln2, Wa, Wb, be1_2,
      W_pos1, bp1_2, W_pos2, bp2_2, Wc)

    total = B * N * K
    n_chunks = total // (_NW * _CH)
    mesh = plsc.VectorSubcoreMesh(core_axis_name="c", subcore_axis_name="s")
    nb, cc = pl.kernel(
        functools.partial(_gather_body, n_chunks),
        mesh=mesh,
        out_type=[
            jax.ShapeDtypeStruct((total, D // 2), jnp.int32),
            jax.ShapeDtypeStruct((total, D // 2), jnp.int32),
        ],
        scratch_types=(
            [pltpu.VMEM((n_chunks * _CH,), jnp.int32)] * 2
            + [pltpu.VMEM((_CH, D // 2), jnp.int32)] * 4
            + [pltpu.SemaphoreType.DMA] * 4
        ),
    )(b_c.reshape(B * N, D // 2), c_table,
      gidx.reshape(total), cidx.reshape(total))

    edges = pl.pallas_call(
        _edge_body,
        grid=(B,),
        in_specs=[
            pl.BlockSpec((1, N, D), lambda i: (i, 0, 0)),
            pl.BlockSpec((1, N * K, D // 2), lambda i: (i, 0, 0)),
            pl.BlockSpec((1, N * K, D // 2), lambda i: (i, 0, 0)),
            pl.BlockSpec((D, D), lambda i: (0, 0)),
            pl.BlockSpec((1, D), lambda i: (0, 0)),
        ],
        out_specs=pl.BlockSpec((1, N * K, D), lambda i: (i, 0, 0)),
        out_shape=jax.ShapeDtypeStruct((B, N * K, D), f32),
    )(a_c, nb.reshape(B, N * K, D // 2),
      cc.reshape(B, N * K, D // 2), W_edge2, be2_2)

    return (nodes, edges.reshape(B, N, K, D), adj)
